# baseline (speedup 1.0000x reference)
import functools

import jax
import jax.numpy as jnp
from jax import lax
from jax.experimental import pallas as pl
from jax.experimental.pallas import tpu as pltpu
from jax.experimental.pallas import tpu_sc as plsc

_N = 10000
_D = 128
_G = 64
_E = 320000

_NC = 2
_NS = 16
_NW = _NC * _NS
_BLK = 112
_KI = 90
_EPW = _KI * _BLK
_EPAD = _NW * _EPW
_NPAD = 112
_NP = _N + _NPAD
_RPT = _NP // _NS


def _make_sc_scatter(F, KG):
  G = _KI // KG
  T = G // 2
  assert G % 2 == 0 and _KI % KG == 0
  mesh = plsc.VectorSubcoreMesh(core_axis_name="c", subcore_axis_name="s")

  @functools.partial(
      pl.kernel,
      mesh=mesh,
      compiler_params=pltpu.CompilerParams(use_tc_tiling_on_sc=False),
      out_type=jax.ShapeDtypeStruct((_NC, _NP, F), jnp.float32),
      scratch_types=[
          pltpu.VMEM((2, KG, _BLK), jnp.int32),
          pltpu.VMEM((2, KG, _BLK), jnp.int32),
          pltpu.VMEM((KG, _BLK, F), jnp.float32),
          pltpu.VMEM((KG, _BLK, F), jnp.float32),
          pltpu.VMEM_SHARED((_NP, F), jnp.float32),
          pltpu.VMEM_SHARED((_N, F), jnp.float32),
          pltpu.SemaphoreType.DMA,
          pltpu.SemaphoreType.DMA,
          pltpu.SemaphoreType.DMA,
          pltpu.SemaphoreType.DMA,
          pltpu.SemaphoreType.DMA,
          pltpu.SemaphoreType.DMA,
          pltpu.SemaphoreType.DMA,
          pltpu.SemaphoreType.DMA,
      ],
  )
  def k(eb_hbm, zeros_hbm, m_hbm, out_hbm, src_v, dst_v,
        rows_a, rows_b, agg_sh, m_sh, is_a, is_b, id_a, id_b, gs_a, gs_b,
        ss_a, ss_b):
    cid = lax.axis_index("c")
    sid = lax.axis_index("s")
    wid = sid * _NC + cid

    pltpu.sync_copy(zeros_hbm.at[pl.ds(sid * _RPT, _RPT)],
                    agg_sh.at[pl.ds(sid * _RPT, _RPT)])
    mrp = _N // _NS
    pltpu.sync_copy(m_hbm.at[pl.ds(sid * mrp, mrp)],
                    m_sh.at[pl.ds(sid * mrp, mrp)])
    plsc.subcore_barrier()

    rows = (rows_a, rows_b)
    isem = (is_a, is_b)
    idsem = (id_a, id_b)
    gsem = (gs_a, gs_b)
    ssem = (ss_a, ss_b)

    def fire_idx_src(g, h):
      pltpu.async_copy(eb_hbm.at[0, wid, pl.ds(g * KG, KG)], src_v.at[h],
                       isem[h])

    def wait_idx_src(g, h):
      pltpu.make_async_copy(eb_hbm.at[0, wid, pl.ds(g * KG, KG)],
                            src_v.at[h], isem[h]).wait()

    def fire_idx_dst(g, h):
      pltpu.async_copy(eb_hbm.at[1, wid, pl.ds(g * KG, KG)], dst_v.at[h],
                       idsem[h])

    def wait_idx_dst(g, h):
      pltpu.make_async_copy(eb_hbm.at[1, wid, pl.ds(g * KG, KG)],
                            dst_v.at[h], idsem[h]).wait()

    def fire_gathers(h):
      for b in range(KG):
        pltpu.async_copy(m_sh.at[src_v.at[h, b]], rows[h].at[b], gsem[h])

    def wait_gathers(h):
      for b in range(KG):
        pltpu.make_async_copy(m_sh.at[src_v.at[h, b]], rows[h].at[b],
                              gsem[h]).wait()

    def fire_scatters(h):
      for b in range(KG):
        pltpu.async_copy(rows[h].at[b], agg_sh.at[dst_v.at[h, b]], ssem[h],
                         add=True)

    def wait_scatters(h):
      for b in range(KG):
        pltpu.make_async_copy(rows[h].at[b], agg_sh.at[dst_v.at[h, b]],
                              ssem[h]).wait()

    fire_idx_src(0, 0)
    fire_idx_dst(0, 0)
    wait_idx_src(0, 0)
    fire_gathers(0)
    fire_idx_src(1, 1)

    def pair(t, carry):
      g0 = 2 * t
      g1 = g0 + 1
      wait_gathers(0)

      @pl.when(g0 + 2 < G)
      def _():
        fire_idx_src(g0 + 2, 0)

      wait_idx_dst(g0, 0)
      fire_scatters(0)

      @pl.when(t > 0)
      def _():
        wait_scatters(1)

      fire_idx_dst(g1, 1)
      wait_idx_src(g1, 1)
      fire_gathers(1)
      wait_gathers(1)

      @pl.when(g1 + 2 < G)
      def _():
        fire_idx_src(g1 + 2, 1)

      wait_idx_dst(g1, 1)
      fire_scatters(1)
      wait_scatters(0)

      @pl.when(g0 + 2 < G)
      def _():
        fire_idx_dst(g0 + 2, 0)
        wait_idx_src(g0 + 2, 0)
        fire_gathers(0)

      return carry

    lax.fori_loop(0, T, pair, 0)

    wait_scatters(1)
    plsc.subcore_barrier()

    pltpu.sync_copy(agg_sh.at[pl.ds(sid * _RPT, _RPT)],
                    out_hbm.at[cid, pl.ds(sid * _RPT, _RPT)])

  return k


_N2 = _N // 2
_NP2 = _NP // 2


def _elu(h):
  return jnp.where(h > 0.0, h, jnp.exp(jnp.minimum(h, 0.0)) - 1.0)


def _pair_mm(hd, w):
  k = w.shape[0]
  return jnp.concatenate(
      [jnp.dot(hd[:, :k], w, preferred_element_type=jnp.float32),
       jnp.dot(hd[:, k:], w, preferred_element_type=jnp.float32)], axis=1)


def _bn_elu_paired(pre, g2, bt2, f):
  mu2 = jnp.mean(pre, axis=0, keepdims=True)
  mu = 0.5 * (mu2[:, :f] + mu2[:, f:])
  mut = jnp.concatenate([mu, mu], axis=1)
  dif = pre - mut
  v2 = jnp.mean(dif * dif, axis=0, keepdims=True)
  var = 0.5 * (v2[:, :f] + v2[:, f:])
  vart = jnp.concatenate([var, var], axis=1)
  return _elu(dif * lax.rsqrt(vart + 1e-5) * g2 + bt2)


def _tc_m1(xd, W1rel):
  def body(xd_ref, wr_ref, m_ref):
    m_ref[...] = _pair_mm(xd_ref[...], wr_ref[...])

  return pl.pallas_call(
      body,
      out_shape=jax.ShapeDtypeStruct((_N2, 2 * _G), jnp.float32),
  )(xd, W1rel)


def _tc_aux_x(xd, W1root, b1, W2rel, W2root, W3rel, W3root):
  def body(xd_ref, wo1_ref, b1_ref, wr2_ref, wo2_ref, wr3_ref, wo3_ref,
           r1_ref, p2x_ref, q2x_ref, p3x_ref, q3x_ref):
    xv = xd_ref[...]
    r1_ref[...] = _pair_mm(xv, wo1_ref[...]) + b1_ref[...]
    p2x_ref[...] = _pair_mm(xv, wr2_ref[: _D, :])
    q2x_ref[...] = _pair_mm(xv, wo2_ref[: _D, :])
    p3x_ref[...] = _pair_mm(xv, wr3_ref[: _D, :])
    q3x_ref[...] = _pair_mm(xv, wo3_ref[: _D, :])

  return pl.pallas_call(
      body,
      out_shape=[
          jax.ShapeDtypeStruct((_N2, 2 * _G), jnp.float32),
          jax.ShapeDtypeStruct((_N2, 2 * _G), jnp.float32),
          jax.ShapeDtypeStruct((_N2, 2 * _G), jnp.float32),
          jax.ShapeDtypeStruct((_N2, 2 * _D), jnp.float32),
          jax.ShapeDtypeStruct((_N2, 2 * _D), jnp.float32),
      ],
  )(xd, W1root, b1, W2rel, W2root, W3rel, W3root)


def _tc_stage2(aggp, r1, g1, bt1, p2x, q2x, W2rel, W2root, b2):
  def body(aggp_ref, r1_ref, g_ref, bt_ref, p2x_ref, q2x_ref, wr_ref, wo_ref,
           b_ref, x1_ref, m2_ref, r2_ref):
    pre = aggp_ref[0, : _N2, :] + aggp_ref[1, : _N2, :] + r1_ref[...]
    x1 = _bn_elu_paired(pre, g_ref[...], bt_ref[...], _G)
    x1_ref[...] = x1
    m2_ref[...] = p2x_ref[...] + _pair_mm(x1, wr_ref[_D :, :])
    r2_ref[...] = q2x_ref[...] + _pair_mm(x1, wo_ref[_D :, :]) + b_ref[...]

  return pl.pallas_call(
      body,
      out_shape=[
          jax.ShapeDtypeStruct((_N2, 2 * _G), jnp.float32),
          jax.ShapeDtypeStruct((_N2, 2 * _G), jnp.float32),
          jax.ShapeDtypeStruct((_N2, 2 * _G), jnp.float32),
      ],
  )(aggp, r1, g1, bt1, p2x, q2x, W2rel, W2root, b2)


def _tc_aux_x1(x1, p3x, q3x, W3rel, W3root):
  def body(x1_ref, p3x_ref, q3x_ref, wr_ref, wo_ref, p3_ref, q3_ref):
    x1v = x1_ref[...]
    p3_ref[...] = p3x_ref[...] + _pair_mm(x1v, wr_ref[_D : _D + _G, :])
    q3_ref[...] = q3x_ref[...] + _pair_mm(x1v, wo_ref[_D : _D + _G, :])

  return pl.pallas_call(
      body,
      out_shape=[
          jax.ShapeDtypeStruct((_N2, 2 * _D), jnp.float32),
          jax.ShapeDtypeStruct((_N2, 2 * _D), jnp.float32),
      ],
  )(x1, p3x, q3x, W3rel, W3root)


def _tc_stage3(aggp, r2, g2, bt2, p3, q3, W3rel, W3root, b3):
  def body(aggp_ref, r2_ref, g_ref, bt_ref, p3_ref, q3_ref, wr_ref, wo_ref,
           b_ref, m3a_ref, m3b_ref, r3_ref):
    pre = aggp_ref[0, : _N2, :] + aggp_ref[1, : _N2, :] + r2_ref[...]
    x2 = _bn_elu_paired(pre, g_ref[...], bt_ref[...], _G)
    m3 = p3_ref[...] + _pair_mm(x2, wr_ref[_D + _G :, :])
    m3a_ref[...] = jnp.concatenate(
        [m3[:, : _G], m3[:, _D : _D + _G]], axis=1)
    m3b_ref[...] = jnp.concatenate(
        [m3[:, _G : _D], m3[:, _D + _G :]], axis=1)
    r3_ref[...] = q3_ref[...] + _pair_mm(x2, wo_ref[_D + _G :, :]) + b_ref[...]

  return pl.pallas_call(
      body,
      out_shape=[
          jax.ShapeDtypeStruct((_N2, 2 * _G), jnp.float32),
          jax.ShapeDtypeStruct((_N2, 2 * _G), jnp.float32),
          jax.ShapeDtypeStruct((_N2, 2 * _D), jnp.float32),
      ],
  )(aggp, r2, g2, bt2, p3, q3, W3rel, W3root, b3)


def _tc_stage4(aggpa, aggpb, r3, g3, bt3, xd):
  def body(aggpa_ref, aggpb_ref, r3_ref, g_ref, bt_ref, xd_ref, out_ref):
    a = aggpa_ref[0, : _N2, :] + aggpa_ref[1, : _N2, :]
    b = aggpb_ref[0, : _N2, :] + aggpb_ref[1, : _N2, :]
    pre = jnp.concatenate(
        [a[:, : _G], b[:, : _G], a[:, _G :], b[:, _G :]],
        axis=1) + r3_ref[...]
    x3 = _bn_elu_paired(pre, g_ref[...], bt_ref[...], _D)
    out_ref[...] = xd_ref[...] + 0.2 * x3

  return pl.pallas_call(
      body,
      out_shape=jax.ShapeDtypeStruct((_N2, 2 * _D), jnp.float32),
  )(aggpa, aggpb, r3, g3, bt3, xd)


def kernel(x, edge_index, W1rel, b1rel, W1root, W2rel, b2rel, W2root,
           W3rel, b3rel, W3root, g1, bt1, g2, bt2, g3, bt3):
  npad = _EPAD - _E
  pad_ids = lax.iota(jnp.int32, npad)
  pad2 = jnp.stack([(pad_ids * 97) % _N, _N + (pad_ids % _NPAD)])
  eb = jnp.concatenate([edge_index.astype(jnp.int32), pad2],
                       axis=1).reshape(2, _NW, _KI, _BLK)

  zeros64 = jnp.zeros((_NP, _G), jnp.float32)

  def tile2(v):
    return jnp.concatenate([v, v]).reshape(1, -1)

  b1t, b2t, b3t = tile2(b1rel), tile2(b2rel), tile2(b3rel)
  g1t, bt1t = tile2(g1), tile2(bt1)
  g2t, bt2t = tile2(g2), tile2(bt2)
  g3t, bt3t = tile2(g3), tile2(bt3)

  xd = x.reshape(_N2, 2 * _D)

  def sc_pair(agg, f):
    return agg.reshape(_NC, _NP2, 2 * f)

  sc64 = _make_sc_scatter(_G, 3)

  m1 = _tc_m1(xd, W1rel)
  agg1 = sc_pair(sc64(eb, zeros64, m1.reshape(_N, _G)), _G)
  r1, p2x, q2x, p3x, q3x = _tc_aux_x(xd, W1root, b1t, W2rel, W2root,
                                     W3rel, W3root)
  x1, m2, r2 = _tc_stage2(agg1, r1, g1t, bt1t, p2x, q2x, W2rel, W2root, b2t)
  agg2 = sc_pair(sc64(eb, zeros64, m2.reshape(_N, _G)), _G)
  p3, q3 = _tc_aux_x1(x1, p3x, q3x, W3rel, W3root)
  m3a, m3b, r3 = _tc_stage3(agg2, r2, g2t, bt2t, p3, q3, W3rel, W3root, b3t)
  agg3a = sc_pair(sc64(eb, zeros64, m3a.reshape(_N, _G)), _G)
  agg3b = sc_pair(sc64(eb, zeros64, m3b.reshape(_N, _G)), _G)
  out = _tc_stage4(agg3a, agg3b, r3, g3t, bt3t, xd)
  return out.reshape(_N, _D)


# speedup vs baseline: 1.0710x; 1.0710x over previous
import functools

import jax
import jax.numpy as jnp
from jax import lax
from jax.experimental import pallas as pl
from jax.experimental.pallas import tpu as pltpu
from jax.experimental.pallas import tpu_sc as plsc

_N = 10000
_D = 128
_G = 64
_E = 320000

_NC = 2
_NS = 16
_NW = _NC * _NS
_BLK = 128
_KI = 80
_EPW = _KI * _BLK
_EPAD = _NW * _EPW
_NPAD = 112
_NP = _N + _NPAD
_RPT = _NP // _NS


def _make_sc_scatter(F, KG):
  G = _KI // KG
  T = G // 2
  assert G % 2 == 0 and _KI % KG == 0
  mesh = plsc.VectorSubcoreMesh(core_axis_name="c", subcore_axis_name="s")

  @functools.partial(
      pl.kernel,
      mesh=mesh,
      compiler_params=pltpu.CompilerParams(use_tc_tiling_on_sc=False),
      out_type=jax.ShapeDtypeStruct((_NC, _NP, F), jnp.float32),
      scratch_types=[
          pltpu.VMEM((2, KG, _BLK), jnp.int32),
          pltpu.VMEM((2, KG, _BLK), jnp.int32),
          pltpu.VMEM((KG, _BLK, F), jnp.float32),
          pltpu.VMEM((KG, _BLK, F), jnp.float32),
          pltpu.VMEM_SHARED((_NP, F), jnp.float32),
          pltpu.SemaphoreType.DMA,
          pltpu.SemaphoreType.DMA,
          pltpu.SemaphoreType.DMA,
          pltpu.SemaphoreType.DMA,
          pltpu.SemaphoreType.DMA,
          pltpu.SemaphoreType.DMA,
          pltpu.SemaphoreType.DMA,
          pltpu.SemaphoreType.DMA,
      ],
  )
  def k(eb_hbm, zeros_hbm, m_hbm, out_hbm, src_v, dst_v,
        rows_a, rows_b, agg_sh, is_a, is_b, id_a, id_b, gs_a, gs_b,
        ss_a, ss_b):
    cid = lax.axis_index("c")
    sid = lax.axis_index("s")
    wid = sid * _NC + cid

    pltpu.sync_copy(zeros_hbm.at[pl.ds(sid * _RPT, _RPT)],
                    agg_sh.at[pl.ds(sid * _RPT, _RPT)])
    plsc.subcore_barrier()

    rows = (rows_a, rows_b)
    isem = (is_a, is_b)
    idsem = (id_a, id_b)
    gsem = (gs_a, gs_b)
    ssem = (ss_a, ss_b)

    def fire_idx_src(g, h):
      pltpu.async_copy(eb_hbm.at[0, wid, pl.ds(g * KG, KG)], src_v.at[h],
                       isem[h])

    def wait_idx_src(g, h):
      pltpu.make_async_copy(eb_hbm.at[0, wid, pl.ds(g * KG, KG)],
                            src_v.at[h], isem[h]).wait()

    def fire_idx_dst(g, h):
      pltpu.async_copy(eb_hbm.at[1, wid, pl.ds(g * KG, KG)], dst_v.at[h],
                       idsem[h])

    def wait_idx_dst(g, h):
      pltpu.make_async_copy(eb_hbm.at[1, wid, pl.ds(g * KG, KG)],
                            dst_v.at[h], idsem[h]).wait()

    def fire_gathers(h):
      for b in range(KG):
        pltpu.async_copy(m_hbm.at[src_v.at[h, b]], rows[h].at[b], gsem[h])

    def wait_gathers(h):
      for b in range(KG):
        pltpu.make_async_copy(m_hbm.at[src_v.at[h, b]], rows[h].at[b],
                              gsem[h]).wait()

    def fire_scatters(h):
      for b in range(KG):
        pltpu.async_copy(rows[h].at[b], agg_sh.at[dst_v.at[h, b]], ssem[h],
                         add=True)

    def wait_scatters(h):
      for b in range(KG):
        pltpu.make_async_copy(rows[h].at[b], agg_sh.at[dst_v.at[h, b]],
                              ssem[h]).wait()

    fire_idx_src(0, 0)
    fire_idx_dst(0, 0)
    wait_idx_src(0, 0)
    fire_gathers(0)
    fire_idx_src(1, 1)

    def pair(t, carry):
      g0 = 2 * t
      g1 = g0 + 1
      wait_gathers(0)

      @pl.when(g0 + 2 < G)
      def _():
        fire_idx_src(g0 + 2, 0)

      wait_idx_dst(g0, 0)
      fire_scatters(0)

      @pl.when(t > 0)
      def _():
        wait_scatters(1)

      fire_idx_dst(g1, 1)
      wait_idx_src(g1, 1)
      fire_gathers(1)
      wait_gathers(1)

      @pl.when(g1 + 2 < G)
      def _():
        fire_idx_src(g1 + 2, 1)

      wait_idx_dst(g1, 1)
      fire_scatters(1)
      wait_scatters(0)

      @pl.when(g0 + 2 < G)
      def _():
        fire_idx_dst(g0 + 2, 0)
        wait_idx_src(g0 + 2, 0)
        fire_gathers(0)

      return carry

    lax.fori_loop(0, T, pair, 0)

    wait_scatters(1)
    plsc.subcore_barrier()

    pltpu.sync_copy(agg_sh.at[pl.ds(sid * _RPT, _RPT)],
                    out_hbm.at[cid, pl.ds(sid * _RPT, _RPT)])

  return k


_N2 = _N // 2
_NP2 = _NP // 2


def _elu(h):
  return jnp.where(h > 0.0, h, jnp.exp(jnp.minimum(h, 0.0)) - 1.0)


def _pair_mm(hd, w):
  k = w.shape[0]
  return jnp.concatenate(
      [jnp.dot(hd[:, :k], w, preferred_element_type=jnp.float32),
       jnp.dot(hd[:, k:], w, preferred_element_type=jnp.float32)], axis=1)


def _bn_elu_paired(pre, g2, bt2, f):
  mu2 = jnp.mean(pre, axis=0, keepdims=True)
  mu = 0.5 * (mu2[:, :f] + mu2[:, f:])
  mut = jnp.concatenate([mu, mu], axis=1)
  dif = pre - mut
  v2 = jnp.mean(dif * dif, axis=0, keepdims=True)
  var = 0.5 * (v2[:, :f] + v2[:, f:])
  vart = jnp.concatenate([var, var], axis=1)
  return _elu(dif * lax.rsqrt(vart + 1e-5) * g2 + bt2)


def _tc_m1(xd, W1rel):
  def body(xd_ref, wr_ref, m_ref):
    m_ref[...] = _pair_mm(xd_ref[...], wr_ref[...])

  return pl.pallas_call(
      body,
      out_shape=jax.ShapeDtypeStruct((_N2, 2 * _G), jnp.float32),
  )(xd, W1rel)


def _tc_aux_x(xd, W1root, b1, W2rel, W2root, W3rel, W3root):
  def body(xd_ref, wo1_ref, b1_ref, wr2_ref, wo2_ref, wr3_ref, wo3_ref,
           r1_ref, p2x_ref, q2x_ref, p3x_ref, q3x_ref):
    xv = xd_ref[...]
    r1_ref[...] = _pair_mm(xv, wo1_ref[...]) + b1_ref[...]
    p2x_ref[...] = _pair_mm(xv, wr2_ref[: _D, :])
    q2x_ref[...] = _pair_mm(xv, wo2_ref[: _D, :])
    p3x_ref[...] = _pair_mm(xv, wr3_ref[: _D, :])
    q3x_ref[...] = _pair_mm(xv, wo3_ref[: _D, :])

  return pl.pallas_call(
      body,
      out_shape=[
          jax.ShapeDtypeStruct((_N2, 2 * _G), jnp.float32),
          jax.ShapeDtypeStruct((_N2, 2 * _G), jnp.float32),
          jax.ShapeDtypeStruct((_N2, 2 * _G), jnp.float32),
          jax.ShapeDtypeStruct((_N2, 2 * _D), jnp.float32),
          jax.ShapeDtypeStruct((_N2, 2 * _D), jnp.float32),
      ],
  )(xd, W1root, b1, W2rel, W2root, W3rel, W3root)


def _tc_stage2(aggp, r1, g1, bt1, p2x, q2x, W2rel, W2root, b2):
  def body(aggp_ref, r1_ref, g_ref, bt_ref, p2x_ref, q2x_ref, wr_ref, wo_ref,
           b_ref, x1_ref, m2_ref, r2_ref):
    pre = aggp_ref[0, : _N2, :] + aggp_ref[1, : _N2, :] + r1_ref[...]
    x1 = _bn_elu_paired(pre, g_ref[...], bt_ref[...], _G)
    x1_ref[...] = x1
    m2_ref[...] = p2x_ref[...] + _pair_mm(x1, wr_ref[_D :, :])
    r2_ref[...] = q2x_ref[...] + _pair_mm(x1, wo_ref[_D :, :]) + b_ref[...]

  return pl.pallas_call(
      body,
      out_shape=[
          jax.ShapeDtypeStruct((_N2, 2 * _G), jnp.float32),
          jax.ShapeDtypeStruct((_N2, 2 * _G), jnp.float32),
          jax.ShapeDtypeStruct((_N2, 2 * _G), jnp.float32),
      ],
  )(aggp, r1, g1, bt1, p2x, q2x, W2rel, W2root, b2)


def _tc_aux_x1(x1, p3x, q3x, W3rel, W3root):
  def body(x1_ref, p3x_ref, q3x_ref, wr_ref, wo_ref, p3_ref, q3_ref):
    x1v = x1_ref[...]
    p3_ref[...] = p3x_ref[...] + _pair_mm(x1v, wr_ref[_D : _D + _G, :])
    q3_ref[...] = q3x_ref[...] + _pair_mm(x1v, wo_ref[_D : _D + _G, :])

  return pl.pallas_call(
      body,
      out_shape=[
          jax.ShapeDtypeStruct((_N2, 2 * _D), jnp.float32),
          jax.ShapeDtypeStruct((_N2, 2 * _D), jnp.float32),
      ],
  )(x1, p3x, q3x, W3rel, W3root)


def _tc_stage3(aggp, r2, g2, bt2, p3, q3, W3rel, W3root, b3):
  def body(aggp_ref, r2_ref, g_ref, bt_ref, p3_ref, q3_ref, wr_ref, wo_ref,
           b_ref, m3a_ref, m3b_ref, r3_ref):
    pre = aggp_ref[0, : _N2, :] + aggp_ref[1, : _N2, :] + r2_ref[...]
    x2 = _bn_elu_paired(pre, g_ref[...], bt_ref[...], _G)
    m3 = p3_ref[...] + _pair_mm(x2, wr_ref[_D + _G :, :])
    m3a_ref[...] = jnp.concatenate(
        [m3[:, : _G], m3[:, _D : _D + _G]], axis=1)
    m3b_ref[...] = jnp.concatenate(
        [m3[:, _G : _D], m3[:, _D + _G :]], axis=1)
    r3_ref[...] = q3_ref[...] + _pair_mm(x2, wo_ref[_D + _G :, :]) + b_ref[...]

  return pl.pallas_call(
      body,
      out_shape=[
          jax.ShapeDtypeStruct((_N2, 2 * _G), jnp.float32),
          jax.ShapeDtypeStruct((_N2, 2 * _G), jnp.float32),
          jax.ShapeDtypeStruct((_N2, 2 * _D), jnp.float32),
      ],
  )(aggp, r2, g2, bt2, p3, q3, W3rel, W3root, b3)


def _tc_stage4(aggpa, aggpb, r3, g3, bt3, xd):
  def body(aggpa_ref, aggpb_ref, r3_ref, g_ref, bt_ref, xd_ref, out_ref):
    a = aggpa_ref[0, : _N2, :] + aggpa_ref[1, : _N2, :]
    b = aggpb_ref[0, : _N2, :] + aggpb_ref[1, : _N2, :]
    pre = jnp.concatenate(
        [a[:, : _G], b[:, : _G], a[:, _G :], b[:, _G :]],
        axis=1) + r3_ref[...]
    x3 = _bn_elu_paired(pre, g_ref[...], bt_ref[...], _D)
    out_ref[...] = xd_ref[...] + 0.2 * x3

  return pl.pallas_call(
      body,
      out_shape=jax.ShapeDtypeStruct((_N2, 2 * _D), jnp.float32),
  )(aggpa, aggpb, r3, g3, bt3, xd)


def kernel(x, edge_index, W1rel, b1rel, W1root, W2rel, b2rel, W2root,
           W3rel, b3rel, W3root, g1, bt1, g2, bt2, g3, bt3):
  npad = _EPAD - _E
  pad_ids = lax.iota(jnp.int32, npad)
  pad2 = jnp.stack([(pad_ids * 97) % _N, _N + (pad_ids % _NPAD)])
  eb = jnp.concatenate([edge_index.astype(jnp.int32), pad2],
                       axis=1).reshape(2, _NW, _KI, _BLK)

  zeros64 = jnp.zeros((_NP, _G), jnp.float32)

  def tile2(v):
    return jnp.concatenate([v, v]).reshape(1, -1)

  b1t, b2t, b3t = tile2(b1rel), tile2(b2rel), tile2(b3rel)
  g1t, bt1t = tile2(g1), tile2(bt1)
  g2t, bt2t = tile2(g2), tile2(bt2)
  g3t, bt3t = tile2(g3), tile2(bt3)

  xd = x.reshape(_N2, 2 * _D)

  def sc_pair(agg, f):
    return agg.reshape(_NC, _NP2, 2 * f)

  sc64 = _make_sc_scatter(_G, 5)

  m1 = _tc_m1(xd, W1rel)
  agg1 = sc_pair(sc64(eb, zeros64, m1.reshape(_N, _G)), _G)
  r1, p2x, q2x, p3x, q3x = _tc_aux_x(xd, W1root, b1t, W2rel, W2root,
                                     W3rel, W3root)
  x1, m2, r2 = _tc_stage2(agg1, r1, g1t, bt1t, p2x, q2x, W2rel, W2root, b2t)
  agg2 = sc_pair(sc64(eb, zeros64, m2.reshape(_N, _G)), _G)
  p3, q3 = _tc_aux_x1(x1, p3x, q3x, W3rel, W3root)
  m3a, m3b, r3 = _tc_stage3(agg2, r2, g2t, bt2t, p3, q3, W3rel, W3root, b3t)
  agg3a = sc_pair(sc64(eb, zeros64, m3a.reshape(_N, _G)), _G)
  agg3b = sc_pair(sc64(eb, zeros64, m3b.reshape(_N, _G)), _G)
  out = _tc_stage4(agg3a, agg3b, r3, g3t, bt3t, xd)
  return out.reshape(_N, _D)
